# baseline (device time: 23314 ns/iter reference)
import jax
import jax.numpy as jnp
from jax import lax
from jax.experimental import pallas as pl
from jax.experimental.pallas import tpu as pltpu

N_DEV = 32
BLK = 32
K = 1024
N_OUT = 1024

_SEND_ORDER = sorted(range(1, N_DEV), key=lambda o: -min(o, N_DEV - o))


def kernel(x, w_mat):
    def body(
        x_ref,
        w_hbm,
        out_ref,
        w_ref,
        x3_ref,
        send_sems,
        recv_sems,
        dsems,
        w_sem,
    ):
        me = lax.axis_index("i")

        wcopy = pltpu.make_async_copy(w_hbm, w_ref, w_sem)
        wcopy.start()

        x3_ref[me] = x_ref[pl.ds(me * BLK, BLK), :]

        barrier_sem = pltpu.get_barrier_semaphore()
        for r, step in enumerate((1, 2, 4, 8, 16)):
            sem = barrier_sem if r == 0 else dsems.at[r - 1]
            pl.semaphore_signal(
                sem,
                inc=1,
                device_id=(lax.rem(me + step, N_DEV),),
                device_id_type=pl.DeviceIdType.MESH,
            )
            pl.semaphore_wait(sem, 1)

        def desc(off):
            dst = lax.rem(me + off, N_DEV)
            return pltpu.make_async_remote_copy(
                src_ref=x_ref.at[pl.ds(dst * BLK, BLK), :],
                dst_ref=x3_ref.at[me],
                send_sem=send_sems.at[off],
                recv_sem=recv_sems.at[off],
                device_id=(dst,),
                device_id_type=pl.DeviceIdType.MESH,
            )

        def rdesc(off):
            src = lax.rem(me + N_DEV - off, N_DEV)
            return pltpu.make_async_remote_copy(
                src_ref=x_ref.at[pl.ds(src * BLK, BLK), :],
                dst_ref=x3_ref.at[src],
                send_sem=send_sems.at[off],
                recv_sem=recv_sems.at[off],
                device_id=(src,),
                device_id_type=pl.DeviceIdType.MESH,
            )

        for off in _SEND_ORDER:
            desc(off).start()
        for off in range(1, N_DEV):
            rdesc(off).wait_recv()
        for off in range(1, N_DEV):
            desc(off).wait_send()

        xr = jnp.transpose(x3_ref[...], (1, 0, 2)).reshape(BLK, K)
        wcopy.wait()
        out_ref[...] = jnp.maximum(
            jnp.dot(xr, w_ref[...], preferred_element_type=jnp.float32), 0.0
        )

    return pl.pallas_call(
        body,
        out_shape=jax.ShapeDtypeStruct((BLK, N_OUT), jnp.float32),
        in_specs=[
            pl.BlockSpec(memory_space=pltpu.VMEM),
            pl.BlockSpec(memory_space=pltpu.MemorySpace.HBM),
        ],
        out_specs=pl.BlockSpec(memory_space=pltpu.VMEM),
        scratch_shapes=[
            pltpu.VMEM((K, N_OUT), jnp.float32),
            pltpu.VMEM((N_DEV, BLK, BLK), jnp.float32),
            pltpu.SemaphoreType.DMA((N_DEV,)),
            pltpu.SemaphoreType.DMA((N_DEV,)),
            pltpu.SemaphoreType.REGULAR((4,)),
            pltpu.SemaphoreType.DMA,
        ],
        compiler_params=pltpu.CompilerParams(collective_id=0),
    )(x, w_mat)


# device time: 21949 ns/iter; 1.0622x vs baseline; 1.0622x over previous
import jax
import jax.numpy as jnp
from jax import lax
from jax.experimental import pallas as pl
from jax.experimental.pallas import tpu as pltpu

N_DEV = 32
BLK = 32
K = 1024
N_OUT = 1024

_SEND_ORDER = sorted(range(1, N_DEV), key=lambda o: -min(o, N_DEV - o))


def kernel(x, w_mat):
    def body(
        x_ref,
        w_hbm,
        out_ref,
        w_ref,
        x3_ref,
        send_sems,
        recv_sems,
        cred_buf,
        cred_send,
        cred_recv,
        w_sem,
    ):
        me = lax.axis_index("i")

        wcopy = pltpu.make_async_copy(w_hbm, w_ref, w_sem)
        wcopy.start()

        x3_ref[me] = x_ref[pl.ds(me * BLK, BLK), :]

        barrier_sem = pltpu.get_barrier_semaphore()
        for off in (1, N_DEV - 1):
            pl.semaphore_signal(
                barrier_sem,
                inc=1,
                device_id=(lax.rem(me + off, N_DEV),),
                device_id_type=pl.DeviceIdType.MESH,
            )
        pl.semaphore_wait(barrier_sem, 2)

        def cdesc(off):
            dst = lax.rem(me + off, N_DEV)
            return pltpu.make_async_remote_copy(
                src_ref=x_ref.at[pl.ds(0, 1), :],
                dst_ref=cred_buf.at[me],
                send_sem=cred_send.at[off],
                recv_sem=cred_recv.at[off],
                device_id=(dst,),
                device_id_type=pl.DeviceIdType.MESH,
            )

        def crdesc(off):
            src = lax.rem(me + N_DEV - off, N_DEV)
            return pltpu.make_async_remote_copy(
                src_ref=x_ref.at[pl.ds(0, 1), :],
                dst_ref=cred_buf.at[src],
                send_sem=cred_send.at[off],
                recv_sem=cred_recv.at[off],
                device_id=(src,),
                device_id_type=pl.DeviceIdType.MESH,
            )

        for off in _SEND_ORDER:
            cdesc(off).start()

        def desc(off):
            dst = lax.rem(me + off, N_DEV)
            return pltpu.make_async_remote_copy(
                src_ref=x_ref.at[pl.ds(dst * BLK, BLK), :],
                dst_ref=x3_ref.at[me],
                send_sem=send_sems.at[off],
                recv_sem=recv_sems.at[off],
                device_id=(dst,),
                device_id_type=pl.DeviceIdType.MESH,
            )

        def rdesc(off):
            src = lax.rem(me + N_DEV - off, N_DEV)
            return pltpu.make_async_remote_copy(
                src_ref=x_ref.at[pl.ds(src * BLK, BLK), :],
                dst_ref=x3_ref.at[src],
                send_sem=send_sems.at[off],
                recv_sem=recv_sems.at[off],
                device_id=(src,),
                device_id_type=pl.DeviceIdType.MESH,
            )

        for off in reversed(_SEND_ORDER):
            crdesc(N_DEV - off).wait_recv()
            desc(off).start()
        for off in range(1, N_DEV):
            rdesc(off).wait_recv()
        for off in range(1, N_DEV):
            desc(off).wait_send()
        for off in range(1, N_DEV):
            cdesc(off).wait_send()

        xr = jnp.transpose(x3_ref[...], (1, 0, 2)).reshape(BLK, K)
        wcopy.wait()
        out_ref[...] = jnp.maximum(
            jnp.dot(xr, w_ref[...], preferred_element_type=jnp.float32), 0.0
        )

    return pl.pallas_call(
        body,
        out_shape=jax.ShapeDtypeStruct((BLK, N_OUT), jnp.float32),
        in_specs=[
            pl.BlockSpec(memory_space=pltpu.VMEM),
            pl.BlockSpec(memory_space=pltpu.MemorySpace.HBM),
        ],
        out_specs=pl.BlockSpec(memory_space=pltpu.VMEM),
        scratch_shapes=[
            pltpu.VMEM((K, N_OUT), jnp.float32),
            pltpu.VMEM((N_DEV, BLK, BLK), jnp.float32),
            pltpu.SemaphoreType.DMA((N_DEV,)),
            pltpu.SemaphoreType.DMA((N_DEV,)),
            pltpu.VMEM((N_DEV, 1, BLK), jnp.float32),
            pltpu.SemaphoreType.DMA((N_DEV,)),
            pltpu.SemaphoreType.DMA((N_DEV,)),
            pltpu.SemaphoreType.DMA,
        ],
        compiler_params=pltpu.CompilerParams(collective_id=0),
    )(x, w_mat)


# device time: 19796 ns/iter; 1.1777x vs baseline; 1.1088x over previous
import jax
import jax.numpy as jnp
from jax import lax
from jax.experimental import pallas as pl
from jax.experimental.pallas import tpu as pltpu

N_DEV = 32
BLK = 32
K = 1024
N_OUT = 1024

_SEND_ORDER = sorted(range(1, N_DEV), key=lambda o: -min(o, N_DEV - o))


def kernel(x, w_mat):
    def body(
        x_ref,
        w_hbm,
        out_ref,
        w_ref,
        x3_ref,
        send_sems,
        recv_sems,
        w_sem,
    ):
        me = lax.axis_index("i")

        wcopy = pltpu.make_async_copy(w_hbm, w_ref, w_sem)
        wcopy.start()

        x3_ref[me] = x_ref[pl.ds(me * BLK, BLK), :]

        barrier_sem = pltpu.get_barrier_semaphore()
        for off in range(1, N_DEV):
            pl.semaphore_signal(
                barrier_sem,
                inc=1,
                device_id=(lax.rem(me + off, N_DEV),),
                device_id_type=pl.DeviceIdType.MESH,
            )
        pl.semaphore_wait(barrier_sem, N_DEV - 1)

        def desc(off):
            dst = lax.rem(me + off, N_DEV)
            return pltpu.make_async_remote_copy(
                src_ref=x_ref.at[pl.ds(dst * BLK, BLK), :],
                dst_ref=x3_ref.at[me],
                send_sem=send_sems.at[off],
                recv_sem=recv_sems.at[off],
                device_id=(dst,),
                device_id_type=pl.DeviceIdType.MESH,
            )

        def rdesc(off):
            src = lax.rem(me + N_DEV - off, N_DEV)
            return pltpu.make_async_remote_copy(
                src_ref=x_ref.at[pl.ds(src * BLK, BLK), :],
                dst_ref=x3_ref.at[src],
                send_sem=send_sems.at[off],
                recv_sem=recv_sems.at[off],
                device_id=(src,),
                device_id_type=pl.DeviceIdType.MESH,
            )

        for off in _SEND_ORDER:
            desc(off).start()
        for off in range(1, N_DEV):
            rdesc(off).wait_recv()
        for off in range(1, N_DEV):
            desc(off).wait_send()

        xr = jnp.transpose(x3_ref[...], (1, 0, 2)).reshape(BLK, K)
        wcopy.wait()
        out_ref[...] = jnp.maximum(
            jnp.dot(xr, w_ref[...], preferred_element_type=jnp.float32), 0.0
        )

    return pl.pallas_call(
        body,
        out_shape=jax.ShapeDtypeStruct((BLK, N_OUT), jnp.float32),
        in_specs=[
            pl.BlockSpec(memory_space=pltpu.VMEM),
            pl.BlockSpec(memory_space=pltpu.MemorySpace.HBM),
        ],
        out_specs=pl.BlockSpec(memory_space=pltpu.VMEM),
        scratch_shapes=[
            pltpu.VMEM((K, N_OUT), jnp.float32),
            pltpu.VMEM((N_DEV, BLK, BLK), jnp.float32),
            pltpu.SemaphoreType.DMA((N_DEV,)),
            pltpu.SemaphoreType.DMA((N_DEV,)),
            pltpu.SemaphoreType.DMA,
        ],
        compiler_params=pltpu.CompilerParams(collective_id=0),
    )(x, w_mat)
